# baseline (speedup 1.0000x reference)
import functools

import jax
import jax.numpy as jnp
from jax import lax
from jax.experimental import pallas as pl
from jax.experimental.pallas import tpu as pltpu
from jax.experimental.pallas import tpu_sc as plsc

B, S, D, E, K = 2, 4096, 2048, 64, 2
TOK_BLK = 1024
N = B * S

_info = plsc.get_sparse_core_info()
NC, NS, L = _info.num_cores, _info.num_subcores, _info.num_lanes
NW = NC * NS
TOK_PER_W = N // NW
GROUPS = TOK_PER_W // L


def _proj_kernel(x_ref, wt_ref, b_ref, h_ref):
    h_ref[...] = (
        jnp.dot(x_ref[...], wt_ref[...], preferred_element_type=jnp.float32)
        + b_ref[...]
    )


def _splat_last(vec):
    idx = jnp.full((L,), L - 1, jnp.int32)
    return lax.gather(
        vec,
        idx[:, None],
        dimension_numbers=lax.GatherDimensionNumbers(
            offset_dims=(), collapsed_slice_dims=(0,), start_index_map=(0,)
        ),
        slice_sizes=(1,),
        mode=lax.GatherScatterMode.PROMISE_IN_BOUNDS,
    )


def _vmax_splat(vec):
    return _splat_last(plsc.cummax(vec))


def _vmin_splat(vec):
    return -_splat_last(plsc.cummax(-vec))


def _route_body(h_hbm, out_hbm, h_v, w_v):
    c = lax.axis_index("c")
    s = lax.axis_index("s")
    wid = s * NC + c
    base = wid * (TOK_PER_W * E)
    pltpu.sync_copy(h_hbm.at[pl.ds(base, TOK_PER_W * E)], h_v)
    lane = lax.iota(jnp.int32, L)
    nj = E // L
    neg_inf = jnp.full((L,), -jnp.inf, jnp.float32)
    one = jnp.full((L,), 1.0, jnp.float32)
    zero = jnp.zeros((L,), jnp.float32)
    big = jnp.full((L,), float(E), jnp.float32)
    eid = [
        (lane + jnp.full((L,), L * j, jnp.int32)).astype(jnp.float32)
        for j in range(nj)
    ]

    def token(t, carry):
        base_t = t * E
        v = [h_v[pl.ds(base_t + L * j, L)] for j in range(nj)]
        m1v = _vmax_splat(jnp.maximum(jnp.maximum(v[0], v[1]),
                                      jnp.maximum(v[2], v[3])))
        t1 = [jnp.where(v[j] == m1v, eid[j], big) for j in range(nj)]
        i1v = _vmin_splat(jnp.minimum(jnp.minimum(t1[0], t1[1]),
                                      jnp.minimum(t1[2], t1[3])))
        sel1 = [t1[j] == i1v for j in range(nj)]
        h2 = [jnp.where(sel1[j], neg_inf, v[j]) for j in range(nj)]
        m2v = _vmax_splat(jnp.maximum(jnp.maximum(h2[0], h2[1]),
                                      jnp.maximum(h2[2], h2[3])))
        t2 = [jnp.where(h2[j] == m2v, eid[j], big) for j in range(nj)]
        i2v = _vmin_splat(jnp.minimum(jnp.minimum(t2[0], t2[1]),
                                      jnp.minimum(t2[2], t2[3])))
        e2 = jnp.exp(m2v - m1v)
        z = one + e2
        w1 = one / z
        w2 = e2 / z
        for j in range(nj):
            w = jnp.where(sel1[j], w1, jnp.where(t2[j] == i2v, w2, zero))
            w_v[pl.ds(base_t + L * j, L)] = w
        return carry

    lax.fori_loop(0, TOK_PER_W, token, 0)
    pltpu.sync_copy(w_v, out_hbm.at[pl.ds(base, TOK_PER_W * E)])


def _route(h_flat):
    mesh = plsc.VectorSubcoreMesh(core_axis_name="c", subcore_axis_name="s")
    return pl.kernel(
        _route_body,
        mesh=mesh,
        out_type=jax.ShapeDtypeStruct((N * E,), jnp.float32),
        compiler_params=pltpu.CompilerParams(needs_layout_passes=False),
        scratch_types=[
            pltpu.VMEM((TOK_PER_W * E,), jnp.float32),
            pltpu.VMEM((TOK_PER_W * E,), jnp.float32),
        ],
    )(h_flat)


@functools.partial(jax.jit, static_argnames=())
def kernel(x, W, b):
    xt = x.reshape(N, D)
    wt = W.T
    b2 = b.reshape(1, E)
    grid = N // TOK_BLK
    h = pl.pallas_call(
        _proj_kernel,
        grid=(grid,),
        in_specs=[
            pl.BlockSpec((TOK_BLK, D), lambda i: (i, 0)),
            pl.BlockSpec((D, E), lambda i: (0, 0)),
            pl.BlockSpec((1, E), lambda i: (0, 0)),
        ],
        out_specs=pl.BlockSpec((TOK_BLK, E), lambda i: (i, 0)),
        out_shape=jax.ShapeDtypeStruct((N, E), jnp.float32),
        compiler_params=pltpu.CompilerParams(
            dimension_semantics=("parallel",),
        ),
    )(xt, wt, b2)
    out = _route(h.reshape(N * E))
    return out.reshape(B, S, E)
